# baseline (speedup 1.0000x reference)
import functools

import jax
import jax.numpy as jnp
from jax import lax
from jax.experimental import pallas as pl
from jax.experimental.pallas import tpu as pltpu
from jax.experimental.pallas import tpu_sc as plsc

_NC = 2
_NS = 16
_NW = _NC * _NS
_LANES = 16


def _make_sc_kernel(B, S, D):
    rows_per_w = S // _NW
    CH = 16
    n_chunks = rows_per_w // CH
    CW = CH * D
    n_vregs = CW // _LANES

    mesh = plsc.VectorSubcoreMesh(core_axis_name="c", subcore_axis_name="s")

    @functools.partial(
        pl.kernel,
        out_type=jax.ShapeDtypeStruct((B, S * D), jnp.float32),
        mesh=mesh,
        scratch_types=[
            pltpu.VMEM((2, B, CW), jnp.float32),
            pltpu.VMEM((2, CW), jnp.float32),
            pltpu.VMEM((2, B, CW), jnp.float32),
            pltpu.SemaphoreType.DMA,
            pltpu.SemaphoreType.DMA,
            pltpu.SemaphoreType.DMA,
            pltpu.SemaphoreType.DMA,
        ],
    )
    def sc_kernel(in_hbm, pos_hbm, out_hbm, in_b, pos_b, out_b, si0, si1, so0, so1):
        wid = lax.axis_index("s") * _NC + lax.axis_index("c")
        base = wid * (rows_per_w * D)
        sin = (si0, si1)
        sout = (so0, so1)

        def start_in(k):
            t = k % 2
            off = base + k * CW
            descs = [pltpu.async_copy(pos_hbm.at[pl.ds(off, CW)], pos_b.at[t], sin[t])]
            for b in range(B):
                descs.append(
                    pltpu.async_copy(in_hbm.at[b, pl.ds(off, CW)], in_b.at[t, b], sin[t])
                )
            return descs

        def start_out(k):
            t = k % 2
            off = base + k * CW
            return [
                pltpu.async_copy(out_b.at[t, b], out_hbm.at[b, pl.ds(off, CW)], sout[t])
                for b in range(B)
            ]

        def compute(k):
            t = k % 2

            @plsc.parallel_loop(0, n_vregs, unroll=4)
            def _(r):
                c = r * _LANES
                po = pos_b[t, pl.ds(c, _LANES)]
                for b in range(B):
                    out_b[t, b, pl.ds(c, _LANES)] = in_b[t, b, pl.ds(c, _LANES)] + po

        pend_in = {0: start_in(0)}
        if n_chunks > 1:
            pend_in[1] = start_in(1)
        pend_out = {}
        for k in range(n_chunks):
            for d in pend_in.pop(k):
                d.wait()
            if k - 2 in pend_out:
                for d in pend_out.pop(k - 2):
                    d.wait()
            compute(k)
            pend_out[k] = start_out(k)
            if k + 2 < n_chunks:
                pend_in[k + 2] = start_in(k + 2)
        for ds_ in pend_out.values():
            for d in ds_:
                d.wait()

    return sc_kernel


def kernel(input_embeddings, pos_table):
    B, S, D = input_embeddings.shape
    flat_in = input_embeddings.reshape(B, S * D)
    flat_pos = pos_table[:S].reshape(S * D)
    out = _make_sc_kernel(B, S, D)(flat_in, flat_pos)
    return out.reshape(B, S, D)


# speedup vs baseline: 1.8927x; 1.8927x over previous
import functools

import jax
import jax.numpy as jnp
from jax import lax
from jax.experimental import pallas as pl
from jax.experimental.pallas import tpu as pltpu
from jax.experimental.pallas import tpu_sc as plsc

_NC = 2
_NS = 16
_NW = _NC * _NS
_LANES = 16


def _make_sc_kernel(B, S, D):
    rows_per_w = S // _NW
    CH = 16
    n_chunks = rows_per_w // CH
    vregs_per_row = D // _LANES

    mesh = plsc.VectorSubcoreMesh(core_axis_name="c", subcore_axis_name="s")

    @functools.partial(
        pl.kernel,
        out_type=jax.ShapeDtypeStruct((B, S, D), jnp.float32),
        mesh=mesh,
        compiler_params=pltpu.CompilerParams(use_tc_tiling_on_sc=True),
        scratch_types=[
            pltpu.VMEM((2, B, CH, D), jnp.float32),
            pltpu.VMEM((2, CH, D), jnp.float32),
            pltpu.VMEM((2, B, CH, D), jnp.float32),
            pltpu.SemaphoreType.DMA((2,)),
            pltpu.SemaphoreType.DMA((2,)),
        ],
    )
    def sc_kernel(in_hbm, pos_hbm, out_hbm, in_b, pos_b, out_b, sin, sout):
        wid = lax.axis_index("s") * _NC + lax.axis_index("c")
        row_base = wid * rows_per_w

        def in_descs(k, t):
            r0 = row_base + k * CH
            descs = [
                pltpu.make_async_copy(pos_hbm.at[pl.ds(r0, CH), :], pos_b.at[t], sin.at[t])
            ]
            for b in range(B):
                descs.append(
                    pltpu.make_async_copy(
                        in_hbm.at[b, pl.ds(r0, CH), :], in_b.at[t, b], sin.at[t]
                    )
                )
            return descs

        def out_descs(k, t):
            r0 = row_base + k * CH
            return [
                pltpu.make_async_copy(
                    out_b.at[t, b], out_hbm.at[b, pl.ds(r0, CH), :], sout.at[t]
                )
                for b in range(B)
            ]

        def start_in(k, t):
            for d in in_descs(k, t):
                d.start()

        def compute(t):
            @plsc.parallel_loop(0, CH)
            def _(row):
                for c in range(vregs_per_row):
                    cs = pl.ds(c * _LANES, _LANES)
                    po = pos_b[t, row, cs]
                    for b in range(B):
                        out_b[t, b, row, cs] = in_b[t, b, row, cs] + po

        start_in(0, 0)
        start_in(1, 1)

        @pl.loop(0, n_chunks)
        def _(k):
            t = lax.rem(k, 2)
            for d in in_descs(k, t):
                d.wait()

            @pl.when(k >= 2)
            def _():
                for d in out_descs(k - 2, t):
                    d.wait()

            compute(t)
            for d in out_descs(k, t):
                d.start()

            @pl.when(k + 2 < n_chunks)
            def _():
                start_in(k + 2, t)

        for k in (n_chunks - 2, n_chunks - 1):
            for d in out_descs(k, k % 2):
                d.wait()

    return sc_kernel


def kernel(input_embeddings, pos_table):
    B, S, D = input_embeddings.shape
    return _make_sc_kernel(B, S, D)(input_embeddings, pos_table[:S])
